# baseline (speedup 1.0000x reference)
import functools

import jax
import jax.numpy as jnp
from jax import lax
from jax.experimental import pallas as pl
from jax.experimental.pallas import tpu as pltpu
from jax.experimental.pallas import tpu_sc as plsc

N, E, D, C, G = 10000, 320000, 128, 10, 64
NPAD = 10240
EPAD = 327680
NTILES = 16
NW = 32
EPW = EPAD // NW
CHUNK = 32
ROWS_PER_TILE = NPAD // NTILES


NACC2 = 10112
CPT = EPW // CHUNK
G_CH = 8
NGRP = CPT // G_CH
IB3 = 3 * G_CH
NH = 8
FAH = 6


@functools.cache
def _make_sc_agg(with_deg):
    rpt = NPAD // NTILES
    scratch = [
        pltpu.VMEM((IB3, 2, CHUNK), jnp.int32),
        pltpu.VMEM((NH * CHUNK, D), jnp.float32),
        pltpu.VMEM_SHARED((NPAD, D), jnp.float32),
        pltpu.SemaphoreType.DMA,
        pltpu.SemaphoreType.DMA,
        pltpu.SemaphoreType.DMA,
    ]
    if with_deg:
        scratch.append(pltpu.VMEM((NACC2,), jnp.float32))
    out_type = [jax.ShapeDtypeStruct((2, NPAD, D), jnp.float32)]
    if with_deg:
        out_type.append(jax.ShapeDtypeStruct((NW, NACC2), jnp.float32))
    mesh = plsc.VectorSubcoreMesh(core_axis_name="c", subcore_axis_name="s")

    @functools.partial(
        pl.kernel, mesh=mesh, out_type=out_type, scratch_types=scratch,
        compiler_params=pltpu.CompilerParams(needs_layout_passes=False))
    def sc_agg(y_hbm, ec_hbm, *refs):
        if with_deg:
            agg_out, deg_out, ibuf, rows2, acc_s, gsem, ssem, isem, deg_v = refs
        else:
            agg_out, ibuf, rows2, acc_s, gsem, ssem, isem = refs
            deg_v = None

        c = lax.axis_index("c")
        s = lax.axis_index("s")
        wid = c * NTILES + s
        base_row = s * rpt
        zeros16 = jnp.zeros((16,), jnp.float32)
        ones16 = jnp.ones((16,), jnp.float32)

        def zrow(i, carry):
            for k in range(D // 16):
                rows2[i, pl.ds(k * 16, 16)] = zeros16
            return carry
        lax.fori_loop(0, 128, zrow, 0)
        zsrc = rows2.at[pl.ds(0, 128)]
        for i in range(rpt // 128):
            pltpu.async_copy(
                zsrc, acc_s.at[pl.ds(base_row + i * 128, 128)], gsem)
        for i in range(rpt // 128):
            pltpu.make_async_copy(
                zsrc, acc_s.at[pl.ds(base_row, 128)], gsem).wait()
        if with_deg:
            def zdeg(i, carry):
                deg_v[pl.ds(i * 16, 16)] = zeros16
                return carry
            lax.fori_loop(0, NACC2 // 16, zdeg, 0)
        plsc.subcore_barrier()

        cbase = wid * CPT

        def deg_update(idx_t):
            if with_deg:
                for j in range(CHUNK // 16):
                    idx16 = ibuf[idx_t, 1, pl.ds(j * 16, 16)]
                    plsc.addupdate_scatter(deg_v, [idx16], ones16)

        def load_group(g, third):
            pltpu.async_copy(ec_hbm.at[pl.ds(cbase + g * G_CH, G_CH)],
                             ibuf.at[pl.ds(third * G_CH, G_CH)], isem)

        def fire_gather(cc, idx_t):
            pltpu.async_copy(y_hbm.at[ibuf.at[idx_t, 0]],
                             rows2.at[pl.ds((cc % NH) * CHUNK, CHUNK)], gsem)

        load_group(0, 0)
        load_group(1, 1)
        pltpu.make_async_copy(ec_hbm.at[pl.ds(0, G_CH)],
                              ibuf.at[pl.ds(0, G_CH)], isem).wait()
        pltpu.make_async_copy(ec_hbm.at[pl.ds(0, G_CH)],
                              ibuf.at[pl.ds(0, G_CH)], isem).wait()
        for j in range(FAH):
            fire_gather(j, j)

        def body(t, idx_t):
            rs = rows2.at[pl.ds((t % NH) * CHUNK, CHUNK)]
            pltpu.make_async_copy(y_hbm.at[ibuf.at[idx_t, 0]], rs,
                                  gsem).wait()
            pltpu.async_copy(rs, acc_s.at[ibuf.at[idx_t, 1]], ssem, add=True)

            @pl.when(t >= NH - FAH)
            def _():
                pltpu.make_async_copy(rows2.at[pl.ds(0, CHUNK)],
                                      acc_s.at[ibuf.at[0, 1]], ssem).wait()

            slot = t % G_CH
            @pl.when((slot == FAH - 1) & (t < (NGRP - 2) * G_CH))
            def _():
                third2 = idx_t // G_CH + 2
                third2 = jnp.where(third2 >= 3, third2 - 3, third2)
                load_group(t // G_CH + 2, third2)

            t2 = t + FAH
            idx2 = jnp.where(idx_t + FAH >= IB3, idx_t + FAH - IB3,
                             idx_t + FAH)

            @pl.when((t2 % G_CH == 0) & (t2 >= 2 * G_CH) & (t2 <= CPT - 1))
            def _():
                pltpu.make_async_copy(ec_hbm.at[pl.ds(0, G_CH)],
                                      ibuf.at[pl.ds(0, G_CH)], isem).wait()

            @pl.when(t2 <= CPT - 1)
            def _():
                fire_gather(t2, idx2)

            deg_update(idx_t)
            idx1 = jnp.where(idx_t + 1 >= IB3, 0, idx_t + 1)
            return idx1

        lax.fori_loop(0, CPT, body, jnp.int32(0))
        for _ in range(NH - FAH):
            pltpu.make_async_copy(rows2.at[pl.ds(0, CHUNK)],
                                  acc_s.at[ibuf.at[0, 1]], ssem).wait()

        plsc.subcore_barrier()

        pltpu.sync_copy(acc_s.at[pl.ds(base_row, rpt)],
                        agg_out.at[c, pl.ds(base_row, rpt)])
        if with_deg:
            pltpu.sync_copy(deg_v, deg_out.at[wid])

    return sc_agg


def _sc_agg_deg(y, ec):
    return _make_sc_agg(True)(y, ec)


def _sc_agg(y, ec):
    return _make_sc_agg(False)(y, ec)[0]


def _layer_body(aggp_ref, degt_ref, w1_ref, b_ref, w2_ref, o_ref):
    i = pl.program_id(0)
    blk = aggp_ref.shape[1]
    a = aggp_ref[0] + aggp_ref[1]
    deg = jnp.sum(degt_ref[...], axis=1, keepdims=True)
    inv = 1.0 / jnp.maximum(deg, 1.0)
    h = jnp.dot(a * inv, w1_ref[...], preferred_element_type=jnp.float32)
    h = jnp.maximum(h + b_ref[...], 0.0)
    row = i * blk + lax.broadcasted_iota(jnp.int32, (blk, 1), 0)
    h = jnp.where(row < N, h, 0.0)
    o_ref[...] = jnp.dot(h, w2_ref[...], preferred_element_type=jnp.float32)


def _tc_layer(aggp, degt, w1, b, w2, blk=2048):
    return pl.pallas_call(
        _layer_body,
        grid=(NPAD // blk,),
        in_specs=[
            pl.BlockSpec((2, blk, D), lambda i: (0, i, 0)),
            pl.BlockSpec((blk, NW), lambda i: (i, 0)),
            pl.BlockSpec((D, D), lambda i: (0, 0)),
            pl.BlockSpec((1, D), lambda i: (0, 0)),
            pl.BlockSpec((D, D), lambda i: (0, 0)),
        ],
        out_specs=pl.BlockSpec((blk, D), lambda i: (i, 0)),
        out_shape=jax.ShapeDtypeStruct((NPAD, D), jnp.float32),
    )(aggp, degt, w1, b, w2)


def _final_body(aggp_ref, degt_ref, b_ref, batch_ref, wh_ref, bh_ref,
                o_ref, pool_ref, cnt_ref):
    i = pl.program_id(0)
    blk = aggp_ref.shape[1]
    a = aggp_ref[0] + aggp_ref[1]
    deg = jnp.sum(degt_ref[...], axis=1, keepdims=True)
    inv = 1.0 / jnp.maximum(deg, 1.0)
    h = jnp.maximum(a * inv + b_ref[...], 0.0)
    row = i * blk + lax.broadcasted_iota(jnp.int32, (blk, 1), 0)
    h = jnp.where(row < N, h, 0.0)
    batch = batch_ref[...]
    cols = lax.broadcasted_iota(jnp.int32, (blk, 128), 1)
    onehot = (batch == cols).astype(jnp.float32)

    @pl.when(i == 0)
    def _():
        pool_ref[...] = jnp.zeros_like(pool_ref)
        cnt_ref[...] = jnp.zeros_like(cnt_ref)

    dn = (((0,), (0,)), ((), ()))
    pool_ref[...] += lax.dot_general(onehot, h, dn,
                                     preferred_element_type=jnp.float32)
    cnt_ref[...] += lax.dot_general(onehot, jnp.ones((blk, 1), jnp.float32),
                                    dn, preferred_element_type=jnp.float32)

    pooled = pool_ref[...] / jnp.maximum(cnt_ref[...], 1.0)
    res = jnp.dot(pooled, wh_ref[...],
                  preferred_element_type=jnp.float32) + bh_ref[...]
    o_ref[...] = res[0:G, :]


def _tc_final(aggp, degt, b, batch, wh, bh, blk=1024):
    return pl.pallas_call(
        _final_body,
        grid=(NPAD // blk,),
        in_specs=[
            pl.BlockSpec((2, blk, D), lambda i: (0, i, 0)),
            pl.BlockSpec((blk, NW), lambda i: (i, 0)),
            pl.BlockSpec((1, D), lambda i: (0, 0)),
            pl.BlockSpec((blk, 1), lambda i: (i, 0)),
            pl.BlockSpec((D, C), lambda i: (0, 0)),
            pl.BlockSpec((1, C), lambda i: (0, 0)),
        ],
        out_specs=pl.BlockSpec((G, C), lambda i: (0, 0)),
        out_shape=jax.ShapeDtypeStruct((G, C), jnp.float32),
        scratch_shapes=[
            pltpu.VMEM((128, D), jnp.float32),
            pltpu.VMEM((128, 1), jnp.float32),
        ],
    )(aggp, degt, b, batch, wh, bh)


@jax.jit
def kernel(x, edge_index, batch_idx, W1, b1, W2, b2, Wh, bh):
    x_pad = jnp.pad(x, ((0, NPAD - N), (0, 0)))
    pad_ids = N + (jnp.arange(EPAD - E, dtype=jnp.int32) % (NACC2 - N))
    src = jnp.concatenate([edge_index[0].astype(jnp.int32), pad_ids])
    dst = jnp.concatenate([edge_index[1].astype(jnp.int32), pad_ids])
    ec = jnp.stack([src, dst], 0).reshape(2, EPAD // CHUNK, CHUNK)
    ec = ec.swapaxes(0, 1).astype(jnp.int32)
    batch = jnp.pad(batch_idx, (0, NPAD - N), constant_values=127)
    batch = batch.reshape(NPAD, 1).astype(jnp.int32)
    b1r = b1.reshape(1, D)
    b2r = b2.reshape(1, D)
    bhr = bh.reshape(1, C)

    aggp1, degp = _sc_agg_deg(x_pad, ec)
    degt = jnp.pad(degp.T, ((0, NPAD - NACC2), (0, 0)))
    y2 = _tc_layer(aggp1, degt, W1, b1r, W2)
    aggp2 = _sc_agg(y2, ec)
    out = _tc_final(aggp2, degt, b2r, batch, Wh, bhr)
    return out


# speedup vs baseline: 1.0709x; 1.0709x over previous
import functools

import jax
import jax.numpy as jnp
from jax import lax
from jax.experimental import pallas as pl
from jax.experimental.pallas import tpu as pltpu
from jax.experimental.pallas import tpu_sc as plsc

N, E, D, C, G = 10000, 320000, 128, 10, 64
NPAD = 10240
EPAD = 327680
NTILES = 16
NW = 32
EPW = EPAD // NW
CHUNK = 64
ROWS_PER_TILE = NPAD // NTILES


NACC2 = 10112
CPT = EPW // CHUNK
G_CH = 8
NGRP = CPT // G_CH
IB3 = 3 * G_CH
NH = 4
FAH = 3


@functools.cache
def _make_sc_agg(with_deg):
    rpt = NPAD // NTILES
    scratch = [
        pltpu.VMEM((IB3, 2, CHUNK), jnp.int32),
        pltpu.VMEM((NH * CHUNK, D), jnp.float32),
        pltpu.VMEM_SHARED((NPAD, D), jnp.float32),
        pltpu.SemaphoreType.DMA,
        pltpu.SemaphoreType.DMA,
        pltpu.SemaphoreType.DMA,
    ]
    if with_deg:
        scratch.append(pltpu.VMEM((NACC2,), jnp.float32))
    out_type = [jax.ShapeDtypeStruct((2, NPAD, D), jnp.float32)]
    if with_deg:
        out_type.append(jax.ShapeDtypeStruct((NW, NACC2), jnp.float32))
    mesh = plsc.VectorSubcoreMesh(core_axis_name="c", subcore_axis_name="s")

    @functools.partial(
        pl.kernel, mesh=mesh, out_type=out_type, scratch_types=scratch,
        compiler_params=pltpu.CompilerParams(needs_layout_passes=False))
    def sc_agg(y_hbm, ec_hbm, *refs):
        if with_deg:
            agg_out, deg_out, ibuf, rows2, acc_s, gsem, ssem, isem, deg_v = refs
        else:
            agg_out, ibuf, rows2, acc_s, gsem, ssem, isem = refs
            deg_v = None

        c = lax.axis_index("c")
        s = lax.axis_index("s")
        wid = c * NTILES + s
        base_row = s * rpt
        zeros16 = jnp.zeros((16,), jnp.float32)
        ones16 = jnp.ones((16,), jnp.float32)

        def zrow(i, carry):
            for k in range(D // 16):
                rows2[i, pl.ds(k * 16, 16)] = zeros16
            return carry
        lax.fori_loop(0, 128, zrow, 0)
        zsrc = rows2.at[pl.ds(0, 128)]
        for i in range(rpt // 128):
            pltpu.async_copy(
                zsrc, acc_s.at[pl.ds(base_row + i * 128, 128)], gsem)
        for i in range(rpt // 128):
            pltpu.make_async_copy(
                zsrc, acc_s.at[pl.ds(base_row, 128)], gsem).wait()
        if with_deg:
            def zdeg(i, carry):
                deg_v[pl.ds(i * 16, 16)] = zeros16
                return carry
            lax.fori_loop(0, NACC2 // 16, zdeg, 0)
        plsc.subcore_barrier()

        cbase = wid * CPT

        def deg_update(idx_t):
            if with_deg:
                for j in range(CHUNK // 16):
                    idx16 = ibuf[idx_t, 1, pl.ds(j * 16, 16)]
                    plsc.addupdate_scatter(deg_v, [idx16], ones16)

        def load_group(g, third):
            pltpu.async_copy(ec_hbm.at[pl.ds(cbase + g * G_CH, G_CH)],
                             ibuf.at[pl.ds(third * G_CH, G_CH)], isem)

        def fire_gather(cc, idx_t):
            pltpu.async_copy(y_hbm.at[ibuf.at[idx_t, 0]],
                             rows2.at[pl.ds((cc % NH) * CHUNK, CHUNK)], gsem)

        load_group(0, 0)
        load_group(1, 1)
        pltpu.make_async_copy(ec_hbm.at[pl.ds(0, G_CH)],
                              ibuf.at[pl.ds(0, G_CH)], isem).wait()
        pltpu.make_async_copy(ec_hbm.at[pl.ds(0, G_CH)],
                              ibuf.at[pl.ds(0, G_CH)], isem).wait()
        for j in range(FAH):
            fire_gather(j, j)

        def body(t, idx_t):
            rs = rows2.at[pl.ds((t % NH) * CHUNK, CHUNK)]
            pltpu.make_async_copy(y_hbm.at[ibuf.at[idx_t, 0]], rs,
                                  gsem).wait()
            pltpu.async_copy(rs, acc_s.at[ibuf.at[idx_t, 1]], ssem, add=True)

            @pl.when(t >= NH - FAH)
            def _():
                pltpu.make_async_copy(rows2.at[pl.ds(0, CHUNK)],
                                      acc_s.at[ibuf.at[0, 1]], ssem).wait()

            slot = t % G_CH
            @pl.when((slot == FAH - 1) & (t < (NGRP - 2) * G_CH))
            def _():
                third2 = idx_t // G_CH + 2
                third2 = jnp.where(third2 >= 3, third2 - 3, third2)
                load_group(t // G_CH + 2, third2)

            t2 = t + FAH
            idx2 = jnp.where(idx_t + FAH >= IB3, idx_t + FAH - IB3,
                             idx_t + FAH)

            @pl.when((t2 % G_CH == 0) & (t2 >= 2 * G_CH) & (t2 <= CPT - 1))
            def _():
                pltpu.make_async_copy(ec_hbm.at[pl.ds(0, G_CH)],
                                      ibuf.at[pl.ds(0, G_CH)], isem).wait()

            @pl.when(t2 <= CPT - 1)
            def _():
                fire_gather(t2, idx2)

            deg_update(idx_t)
            idx1 = jnp.where(idx_t + 1 >= IB3, 0, idx_t + 1)
            return idx1

        lax.fori_loop(0, CPT, body, jnp.int32(0))
        for _ in range(NH - FAH):
            pltpu.make_async_copy(rows2.at[pl.ds(0, CHUNK)],
                                  acc_s.at[ibuf.at[0, 1]], ssem).wait()

        plsc.subcore_barrier()

        pltpu.sync_copy(acc_s.at[pl.ds(base_row, rpt)],
                        agg_out.at[c, pl.ds(base_row, rpt)])
        if with_deg:
            pltpu.sync_copy(deg_v, deg_out.at[wid])

    return sc_agg


def _sc_agg_deg(y, ec):
    return _make_sc_agg(True)(y, ec)


def _sc_agg(y, ec):
    return _make_sc_agg(False)(y, ec)[0]


def _layer_body(aggp_ref, degt_ref, w1_ref, b_ref, w2_ref, o_ref):
    i = pl.program_id(0)
    blk = aggp_ref.shape[1]
    a = aggp_ref[0] + aggp_ref[1]
    deg = jnp.sum(degt_ref[...], axis=1, keepdims=True)
    inv = 1.0 / jnp.maximum(deg, 1.0)
    h = jnp.dot(a * inv, w1_ref[...], preferred_element_type=jnp.float32)
    h = jnp.maximum(h + b_ref[...], 0.0)
    row = i * blk + lax.broadcasted_iota(jnp.int32, (blk, 1), 0)
    h = jnp.where(row < N, h, 0.0)
    o_ref[...] = jnp.dot(h, w2_ref[...], preferred_element_type=jnp.float32)


def _tc_layer(aggp, degt, w1, b, w2, blk=2048):
    return pl.pallas_call(
        _layer_body,
        grid=(NPAD // blk,),
        in_specs=[
            pl.BlockSpec((2, blk, D), lambda i: (0, i, 0)),
            pl.BlockSpec((blk, NW), lambda i: (i, 0)),
            pl.BlockSpec((D, D), lambda i: (0, 0)),
            pl.BlockSpec((1, D), lambda i: (0, 0)),
            pl.BlockSpec((D, D), lambda i: (0, 0)),
        ],
        out_specs=pl.BlockSpec((blk, D), lambda i: (i, 0)),
        out_shape=jax.ShapeDtypeStruct((NPAD, D), jnp.float32),
    )(aggp, degt, w1, b, w2)


def _final_body(aggp_ref, degt_ref, b_ref, batch_ref, wh_ref, bh_ref,
                o_ref, pool_ref, cnt_ref):
    i = pl.program_id(0)
    blk = aggp_ref.shape[1]
    a = aggp_ref[0] + aggp_ref[1]
    deg = jnp.sum(degt_ref[...], axis=1, keepdims=True)
    inv = 1.0 / jnp.maximum(deg, 1.0)
    h = jnp.maximum(a * inv + b_ref[...], 0.0)
    row = i * blk + lax.broadcasted_iota(jnp.int32, (blk, 1), 0)
    h = jnp.where(row < N, h, 0.0)
    batch = batch_ref[...]
    cols = lax.broadcasted_iota(jnp.int32, (blk, 128), 1)
    onehot = (batch == cols).astype(jnp.float32)

    @pl.when(i == 0)
    def _():
        pool_ref[...] = jnp.zeros_like(pool_ref)
        cnt_ref[...] = jnp.zeros_like(cnt_ref)

    dn = (((0,), (0,)), ((), ()))
    pool_ref[...] += lax.dot_general(onehot, h, dn,
                                     preferred_element_type=jnp.float32)
    cnt_ref[...] += lax.dot_general(onehot, jnp.ones((blk, 1), jnp.float32),
                                    dn, preferred_element_type=jnp.float32)

    pooled = pool_ref[...] / jnp.maximum(cnt_ref[...], 1.0)
    res = jnp.dot(pooled, wh_ref[...],
                  preferred_element_type=jnp.float32) + bh_ref[...]
    o_ref[...] = res[0:G, :]


def _tc_final(aggp, degt, b, batch, wh, bh, blk=1024):
    return pl.pallas_call(
        _final_body,
        grid=(NPAD // blk,),
        in_specs=[
            pl.BlockSpec((2, blk, D), lambda i: (0, i, 0)),
            pl.BlockSpec((blk, NW), lambda i: (i, 0)),
            pl.BlockSpec((1, D), lambda i: (0, 0)),
            pl.BlockSpec((blk, 1), lambda i: (i, 0)),
            pl.BlockSpec((D, C), lambda i: (0, 0)),
            pl.BlockSpec((1, C), lambda i: (0, 0)),
        ],
        out_specs=pl.BlockSpec((G, C), lambda i: (0, 0)),
        out_shape=jax.ShapeDtypeStruct((G, C), jnp.float32),
        scratch_shapes=[
            pltpu.VMEM((128, D), jnp.float32),
            pltpu.VMEM((128, 1), jnp.float32),
        ],
    )(aggp, degt, b, batch, wh, bh)


@jax.jit
def kernel(x, edge_index, batch_idx, W1, b1, W2, b2, Wh, bh):
    x_pad = jnp.pad(x, ((0, NPAD - N), (0, 0)))
    pad_ids = N + (jnp.arange(EPAD - E, dtype=jnp.int32) % (NACC2 - N))
    src = jnp.concatenate([edge_index[0].astype(jnp.int32), pad_ids])
    dst = jnp.concatenate([edge_index[1].astype(jnp.int32), pad_ids])
    ec = jnp.stack([src, dst], 0).reshape(2, EPAD // CHUNK, CHUNK)
    ec = ec.swapaxes(0, 1).astype(jnp.int32)
    batch = jnp.pad(batch_idx, (0, NPAD - N), constant_values=127)
    batch = batch.reshape(NPAD, 1).astype(jnp.int32)
    b1r = b1.reshape(1, D)
    b2r = b2.reshape(1, D)
    bhr = bh.reshape(1, C)

    aggp1, degp = _sc_agg_deg(x_pad, ec)
    degt = jnp.pad(degp.T, ((0, NPAD - NACC2), (0, 0)))
    y2 = _tc_layer(aggp1, degt, W1, b1r, W2)
    aggp2 = _sc_agg(y2, ec)
    out = _tc_final(aggp2, degt, b2r, batch, Wh, bhr)
    return out
